# baseline (speedup 1.0000x reference)
import functools

import numpy as np
import jax
import jax.numpy as jnp
from jax import lax
from jax.experimental import pallas as pl
from jax.experimental.pallas import tpu as pltpu
from jax.experimental.pallas import tpu_sc as plsc

_N = 10000
_E = 320000
_EIN = 16
_H = 128
_L = 4
_MASK_RATIO = 0.15

_NC = 2
_NS = 16
_NW = _NC * _NS
_EPW = _E // _NW
_CH = 80
_NCHUNK = _EPW // _CH
_CH1 = 40
_NCHUNK1 = _EPW // _CH1
_ETBL = _E + 4000
_RPT = (_N // _NS) // 8 * 8
_RTAIL = _N - _RPT * _NS

_PERM = (np.arange(16)[None, None, :] * 2
         + np.array([0, 1])[None, :, None]
         + (np.arange(_H // 32) * 32)[:, None, None]).reshape(_H)

_mesh = plsc.VectorSubcoreMesh(core_axis_name="c", subcore_axis_name="s")


@functools.partial(
    pl.kernel,
    out_type=jax.ShapeDtypeStruct((_E, _EIN), jnp.float32),
    mesh=_mesh,
    scratch_types=[
        pltpu.VMEM((_NCHUNK, _CH), jnp.int32),
        pltpu.VMEM((2, _CH, _EIN), jnp.float32),
        pltpu.SemaphoreType.DMA((2,)),
        pltpu.SemaphoreType.DMA((2,)),
    ],
    compiler_params=pltpu.CompilerParams(use_tc_tiling_on_sc=False),
)
def _sc_masked_gather(tbl, ridx3, out, idx_all, buf, gsem, wsem):
    c = lax.axis_index("c")
    s = lax.axis_index("s")
    wid = s * _NC + c
    base = wid * _EPW

    pltpu.sync_copy(ridx3.at[wid], idx_all)
    pltpu.async_copy(tbl.at[idx_all.at[0]], buf.at[0], gsem.at[0])

    def chunk(g, carry):
        par = lax.rem(g, 2)
        npar = lax.rem(g + 1, 2)

        @pl.when(g + 1 < _NCHUNK)
        def _():
            @pl.when(g >= 1)
            def _():
                pltpu.make_async_copy(
                    buf.at[npar], out.at[pl.ds(base, _CH)], wsem.at[npar]
                ).wait()

            pltpu.async_copy(tbl.at[idx_all.at[g + 1]], buf.at[npar],
                             gsem.at[npar])

        pltpu.make_async_copy(tbl.at[idx_all.at[g]], buf.at[par],
                              gsem.at[par]).wait()
        pltpu.async_copy(buf.at[par], out.at[pl.ds(base + g * _CH, _CH)],
                         wsem.at[par])
        return carry

    lax.fori_loop(0, _NCHUNK, chunk, 0)
    pltpu.make_async_copy(buf.at[0], out.at[pl.ds(base, _CH)],
                          wsem.at[0]).wait()
    pltpu.make_async_copy(buf.at[1], out.at[pl.ds(base, _CH)],
                          wsem.at[1]).wait()


def _widen(packed):
    lo = plsc.bitcast(packed << 16, jnp.float32)
    hi = plsc.bitcast(packed & jnp.int32(-65536), jnp.float32)
    return lo, hi


@functools.partial(
    pl.kernel,
    out_type=jax.ShapeDtypeStruct((_NC, _N, _H), jnp.float32),
    mesh=_mesh,
    scratch_types=[
        pltpu.VMEM((3, _CH), jnp.int32),
        pltpu.VMEM((3, _CH), jnp.int32),
        pltpu.VMEM((2, _CH, _H // 2), jnp.int32),
        pltpu.VMEM((2, _CH, _H // 2), jnp.int32),
        pltpu.VMEM((2, _CH, _H), jnp.float32),
        pltpu.VMEM_SHARED((_N, _H), jnp.float32),
        pltpu.SemaphoreType.DMA((2,)),
        pltpu.SemaphoreType.DMA((2,)),
        pltpu.SemaphoreType.DMA((3,)),
    ],
    compiler_params=pltpu.CompilerParams(use_tc_tiling_on_sc=False,
                                         needs_layout_passes=False),
)
def _sc_message_pass(src, dst, hs, hd, ee, zero, out,
                     sring, dring, bufa, bufb, bufc, acc,
                     gsem, ssem, rsem):
    c = lax.axis_index("c")
    s = lax.axis_index("s")
    wid = s * _NC + c
    base = wid * _EPW

    pltpu.sync_copy(src.at[pl.ds(base, _CH)], sring.at[0])
    pltpu.sync_copy(dst.at[pl.ds(base, _CH)], dring.at[0])

    pltpu.sync_copy(zero.at[pl.ds(s * _RPT, _RPT)],
                    acc.at[pl.ds(s * _RPT, _RPT)])

    @pl.when(s == _NS - 1)
    def _():
        pltpu.sync_copy(zero.at[pl.ds(_NS * _RPT, _RTAIL)],
                        acc.at[pl.ds(_NS * _RPT, _RTAIL)])

    plsc.subcore_barrier()

    def issue(g, par, slot):
        pltpu.async_copy(hs.at[sring.at[slot]], bufa.at[par], gsem.at[par])
        pltpu.async_copy(hd.at[dring.at[slot]], bufb.at[par], gsem.at[par])
        pltpu.async_copy(ee.at[pl.ds(base + g * _CH, _CH)], bufc.at[par],
                         gsem.at[par])

    def ring_load(g, slot):
        pltpu.async_copy(src.at[pl.ds(base + g * _CH, _CH)],
                         sring.at[slot], rsem.at[slot])
        pltpu.async_copy(dst.at[pl.ds(base + g * _CH, _CH)],
                         dring.at[slot], rsem.at[slot])

    issue(0, 0, 0)
    ring_load(1, 1)

    def chunk(g, carry):
        par = lax.rem(g, 2)
        npar = lax.rem(g + 1, 2)
        slot = lax.rem(g, 3)
        nslot = lax.rem(g + 1, 3)
        n2slot = lax.rem(g + 2, 3)

        @pl.when(g >= 1)
        def _():
            pltpu.make_async_copy(
                bufc.at[npar], acc.at[dring.at[nslot]], ssem.at[npar]
            ).wait()

        @pl.when(g + 1 < _NCHUNK)
        def _():
            pltpu.make_async_copy(src.at[pl.ds(base, _CH)],
                                  sring.at[nslot], rsem.at[nslot]).wait()
            pltpu.make_async_copy(dst.at[pl.ds(base, _CH)],
                                  dring.at[nslot], rsem.at[nslot]).wait()
            issue(g + 1, npar, nslot)

            @pl.when(g + 2 < _NCHUNK)
            def _():
                ring_load(g + 2, n2slot)

        pltpu.make_async_copy(hs.at[sring.at[slot]], bufa.at[par],
                              gsem.at[par]).wait()
        pltpu.make_async_copy(hd.at[dring.at[slot]], bufb.at[par],
                              gsem.at[par]).wait()
        pltpu.make_async_copy(ee.at[pl.ds(base, _CH)], bufc.at[par],
                              gsem.at[par]).wait()

        def row(i2, rcarry):
            for u in range(2):
                i = i2 * 2 + u
                for j in range(_H // 32):
                    a_lo, a_hi = _widen(bufa[par, i, pl.ds(16 * j, 16)])
                    b_lo, b_hi = _widen(bufb[par, i, pl.ds(16 * j, 16)])
                    slo = pl.ds(32 * j, 16)
                    shi = pl.ds(32 * j + 16, 16)
                    bufc[par, i, slo] = jnp.maximum(
                        a_lo + b_lo + bufc[par, i, slo], 0.0)
                    bufc[par, i, shi] = jnp.maximum(
                        a_hi + b_hi + bufc[par, i, shi], 0.0)
            return rcarry

        lax.fori_loop(0, _CH // 2, row, 0)
        pltpu.async_copy(bufc.at[par], acc.at[dring.at[slot]],
                         ssem.at[par], add=True)
        return carry

    lax.fori_loop(0, _NCHUNK, chunk, 0)

    pltpu.make_async_copy(bufc.at[(_NCHUNK - 1) % 2], acc.at[dring.at[0]],
                          ssem.at[(_NCHUNK - 1) % 2]).wait()
    plsc.subcore_barrier()

    pltpu.sync_copy(acc.at[pl.ds(s * _RPT, _RPT)],
                    out.at[c, pl.ds(s * _RPT, _RPT)])

    @pl.when(s == _NS - 1)
    def _():
        pltpu.sync_copy(acc.at[pl.ds(_NS * _RPT, _RTAIL)],
                        out.at[c, pl.ds(_NS * _RPT, _RTAIL)])


@functools.partial(
    pl.kernel,
    out_type=jax.ShapeDtypeStruct((_E, _H), jnp.float32),
    mesh=_mesh,
    scratch_types=[
        pltpu.VMEM((_EPW,), jnp.int32),
        pltpu.VMEM((_EPW,), jnp.int32),
        pltpu.VMEM((2, _CH, _H), jnp.float32),
        pltpu.VMEM((2, _CH, _H), jnp.float32),
        pltpu.SemaphoreType.DMA((2,)),
        pltpu.SemaphoreType.DMA((2,)),
    ],
    compiler_params=pltpu.CompilerParams(needs_layout_passes=False),
)
def _sc_decoder_gather(src, dst, ns, nd, out,
                       sidx_all, didx_all, bufa, bufb, gsem, wsem):
    c = lax.axis_index("c")
    s = lax.axis_index("s")
    wid = s * _NC + c
    base = wid * _EPW

    pltpu.sync_copy(src.at[pl.ds(base, _EPW)], sidx_all)
    pltpu.sync_copy(dst.at[pl.ds(base, _EPW)], didx_all)

    def issue(g, par):
        pltpu.async_copy(ns.at[sidx_all.at[pl.ds(g * _CH, _CH)]],
                         bufa.at[par], gsem.at[par])
        pltpu.async_copy(nd.at[didx_all.at[pl.ds(g * _CH, _CH)]],
                         bufb.at[par], gsem.at[par])

    issue(0, 0)

    def chunk(g, carry):
        par = lax.rem(g, 2)
        npar = lax.rem(g + 1, 2)

        @pl.when(g + 1 < _NCHUNK)
        def _():
            @pl.when(g >= 1)
            def _():
                pltpu.make_async_copy(
                    bufb.at[npar], out.at[pl.ds(base, _CH)], wsem.at[npar]
                ).wait()

            issue(g + 1, npar)

        pltpu.make_async_copy(ns.at[sidx_all.at[pl.ds(0, _CH)]],
                              bufa.at[par], gsem.at[par]).wait()
        pltpu.make_async_copy(nd.at[didx_all.at[pl.ds(0, _CH)]],
                              bufb.at[par], gsem.at[par]).wait()

        def row(i2, rcarry):
            for u in range(2):
                i = i2 * 2 + u
                for j in range(_H // 16):
                    sl = pl.ds(16 * j, 16)
                    bufb[par, i, sl] = jnp.maximum(
                        bufa[par, i, sl] + bufb[par, i, sl], 0.0)
            return rcarry

        lax.fori_loop(0, _CH // 2, row, 0)
        pltpu.async_copy(bufb.at[par], out.at[pl.ds(base + g * _CH, _CH)],
                         wsem.at[par])
        return carry

    lax.fori_loop(0, _NCHUNK, chunk, 0)
    pltpu.make_async_copy(bufb.at[0], out.at[pl.ds(base, _CH)],
                          wsem.at[0]).wait()
    pltpu.make_async_copy(bufb.at[1], out.at[pl.ds(base, _CH)],
                          wsem.at[1]).wait()


def _linear_relu_body(x_ref, w_ref, b_ref, o_ref):
    acc = jnp.dot(x_ref[...], w_ref[...], preferred_element_type=jnp.float32)
    o_ref[...] = jnp.maximum(acc + b_ref[...], 0.0)


def _tc_linear_relu(x, w, b, bm):
    r, k = x.shape
    o = w.shape[1]
    return pl.pallas_call(
        _linear_relu_body,
        grid=(r // bm,),
        in_specs=[
            pl.BlockSpec((bm, k), lambda i: (i, 0)),
            pl.BlockSpec((k, o), lambda i: (0, 0)),
            pl.BlockSpec((1, o), lambda i: (0, 0)),
        ],
        out_specs=pl.BlockSpec((bm, o), lambda i: (i, 0)),
        out_shape=jax.ShapeDtypeStruct((r, o), jnp.float32),
    )(x, w, b)


def _dual_mm_body(x_ref, ws_ref, wd_ref, bs_ref, os_ref, od_ref):
    xb = x_ref[...]
    os_ref[...] = (
        jnp.dot(xb, ws_ref[...], preferred_element_type=jnp.float32)
        + bs_ref[...]
    ).astype(os_ref.dtype)
    od_ref[...] = jnp.dot(
        xb, wd_ref[...], preferred_element_type=jnp.float32
    ).astype(od_ref.dtype)


def _tc_dual_mm(x, ws, wd, bs, bm, dtype=jnp.bfloat16):
    r, k = x.shape
    o = ws.shape[1]
    return pl.pallas_call(
        _dual_mm_body,
        grid=(r // bm,),
        in_specs=[
            pl.BlockSpec((bm, k), lambda i: (i, 0)),
            pl.BlockSpec((k, o), lambda i: (0, 0)),
            pl.BlockSpec((k, o), lambda i: (0, 0)),
            pl.BlockSpec((1, o), lambda i: (0, 0)),
        ],
        out_specs=[
            pl.BlockSpec((bm, o), lambda i: (i, 0)),
            pl.BlockSpec((bm, o), lambda i: (i, 0)),
        ],
        out_shape=[
            jax.ShapeDtypeStruct((r, o), dtype),
            jax.ShapeDtypeStruct((r, o), dtype),
        ],
    )(x, ws, wd, bs)


def _encode_e_body(ma_ref, tok_ref, we_ref, be_ref, o_ref):
    nblk = _E // 2000
    ma = ma_ref[...]
    ma = jnp.where(pl.program_id(0) < nblk, ma,
                   jnp.broadcast_to(tok_ref[...], ma.shape))
    e = jnp.maximum(
        jnp.dot(ma, we_ref[...], preferred_element_type=jnp.float32)
        + be_ref[...],
        0.0,
    )
    o_ref[...] = e.astype(jnp.bfloat16)


def _tc_encode_e(ea, tok, we, be):
    bm = 2000
    nblk = _E // bm
    return pl.pallas_call(
        _encode_e_body,
        grid=(_ETBL // bm,),
        in_specs=[
            pl.BlockSpec((bm, _EIN), lambda i: (jnp.minimum(i, _E // 2000 - 1), 0)),
            pl.BlockSpec((1, _EIN), lambda i: (0, 0)),
            pl.BlockSpec((_EIN, _H), lambda i: (0, 0)),
            pl.BlockSpec((1, _H), lambda i: (0, 0)),
        ],
        out_specs=pl.BlockSpec((bm, _H), lambda i: (i, 0)),
        out_shape=jax.ShapeDtypeStruct((_ETBL, _H), jnp.bfloat16),
    )(ea, tok, we, be)


def _ee_mm_body(e_ref, wm_ref, bm_ref, o_ref):
    o_ref[...] = (
        jnp.dot(e_ref[...], wm_ref[...], preferred_element_type=jnp.float32)
        + bm_ref[...]
    )


def _tc_ee_mm(e, wm, bmb, bm):
    r = e.shape[0]
    return pl.pallas_call(
        _ee_mm_body,
        grid=(r // bm,),
        in_specs=[
            pl.BlockSpec((bm, _H), lambda i: (i, 0)),
            pl.BlockSpec((_H, _H), lambda i: (0, 0)),
            pl.BlockSpec((1, _H), lambda i: (0, 0)),
        ],
        out_specs=pl.BlockSpec((bm, _H), lambda i: (i, 0)),
        out_shape=jax.ShapeDtypeStruct((r, _H), jnp.float32),
    )(e, wm, bmb)


def _edge_feat_body(ma_ref, we_ref, be_ref, wm_ref, bm_ref, o_ref):
    e = jnp.maximum(
        jnp.dot(ma_ref[...], we_ref[...], preferred_element_type=jnp.float32)
        + be_ref[...],
        0.0,
    )
    o_ref[...] = (
        jnp.dot(e, wm_ref[...], preferred_element_type=jnp.float32)
        + bm_ref[...]
    )


def _tc_edge_feat(ma, we, be, wm, bmb, bm):
    r = ma.shape[0]
    return pl.pallas_call(
        _edge_feat_body,
        grid=(r // bm,),
        in_specs=[
            pl.BlockSpec((bm, _EIN), lambda i: (i, 0)),
            pl.BlockSpec((_EIN, _H), lambda i: (0, 0)),
            pl.BlockSpec((1, _H), lambda i: (0, 0)),
            pl.BlockSpec((_H, _H), lambda i: (0, 0)),
            pl.BlockSpec((1, _H), lambda i: (0, 0)),
        ],
        out_specs=pl.BlockSpec((bm, _H), lambda i: (i, 0)),
        out_shape=jax.ShapeDtypeStruct((r, _H), jnp.float32),
    )(ma, we, be, wm, bmb)


def _update_body(h_ref, a_ref, wh_ref, wa_ref, bu_ref, o_ref):
    hb = h_ref[...]
    ab = a_ref[0] + a_ref[1]
    upd = jnp.maximum(
        jnp.dot(hb, wh_ref[...], preferred_element_type=jnp.float32)
        + jnp.dot(ab, wa_ref[...], preferred_element_type=jnp.float32)
        + bu_ref[...],
        0.0,
    )
    o_ref[...] = hb + upd


def _tc_update(h, aggp, wh, wa, bu, bm):
    r = h.shape[0]
    return pl.pallas_call(
        _update_body,
        grid=(r // bm,),
        in_specs=[
            pl.BlockSpec((bm, _H), lambda i: (i, 0)),
            pl.BlockSpec((_NC, bm, _H), lambda i: (0, i, 0)),
            pl.BlockSpec((_H, _H), lambda i: (0, 0)),
            pl.BlockSpec((_H, _H), lambda i: (0, 0)),
            pl.BlockSpec((1, _H), lambda i: (0, 0)),
        ],
        out_specs=pl.BlockSpec((bm, _H), lambda i: (i, 0)),
        out_shape=jax.ShapeDtypeStruct((r, _H), jnp.float32),
    )(h, aggp, wh, wa, bu)


def _recon_body(hid_ref, ea_ref, mf_ref, w2_ref, b2_ref, rec_ref, lacc_ref):
    rec = (
        jnp.dot(hid_ref[...], w2_ref[...], preferred_element_type=jnp.float32)
        + b2_ref[...]
    )
    rec_ref[...] = rec
    d = (rec - ea_ref[...]) * mf_ref[...]
    part = jnp.sum(d * d)

    @pl.when(pl.program_id(0) == 0)
    def _():
        lacc_ref[0, 0] = 0.0

    lacc_ref[0, 0] += part


def _tc_recon_loss(hid, ea, mf, w2, b2, bm):
    r = hid.shape[0]
    return pl.pallas_call(
        _recon_body,
        grid=(r // bm,),
        in_specs=[
            pl.BlockSpec((bm, _H), lambda i: (i, 0)),
            pl.BlockSpec((bm, _EIN), lambda i: (i, 0)),
            pl.BlockSpec((bm, _EIN), lambda i: (i, 0)),
            pl.BlockSpec((_H, _EIN), lambda i: (0, 0)),
            pl.BlockSpec((1, _EIN), lambda i: (0, 0)),
        ],
        out_specs=[
            pl.BlockSpec((bm, _EIN), lambda i: (i, 0)),
            pl.BlockSpec(memory_space=pltpu.SMEM),
        ],
        out_shape=[
            jax.ShapeDtypeStruct((r, _EIN), jnp.float32),
            jax.ShapeDtypeStruct((1, 1), jnp.float32),
        ],
    )(hid, ea, mf, w2, b2)


_ROT0 = (13, 15, 26, 6)
_ROT1 = (17, 29, 16, 24)


def _rotl(x, d):
    return ((x << np.uint32(d)) | (x >> np.uint32(32 - d))).astype(np.uint32)


def _threefry2x32(k1, k2, x1, x2):
    with np.errstate(over='ignore'):
        ks = (np.uint32(k1), np.uint32(k2),
              np.uint32(k1) ^ np.uint32(k2) ^ np.uint32(0x1BD11BDA))
        x = [x1.astype(np.uint32) + ks[0], x2.astype(np.uint32) + ks[1]]
        rots = (_ROT0, _ROT1, _ROT0, _ROT1, _ROT0)
        kidx = ((1, 2), (2, 0), (0, 1), (1, 2), (2, 0))
        for i in range(5):
            for r in rots[i]:
                x[0] = (x[0] + x[1]).astype(np.uint32)
                x[1] = x[0] ^ _rotl(x[1], r)
            x[0] = (x[0] + ks[kidx[i][0]]).astype(np.uint32)
            x[1] = (x[1] + ks[kidx[i][1]] + np.uint32(i + 1)).astype(np.uint32)
    return x[0], x[1]


def _random_bits32(key, n):
    hi = np.zeros((n,), np.uint32)
    lo = np.arange(n, dtype=np.uint32)
    b1, b2 = _threefry2x32(key[0], key[1], hi, lo)
    return b1 ^ b2


def _np_split(key, num):
    hi = np.zeros((num,), np.uint32)
    lo = np.arange(num, dtype=np.uint32)
    b1, b2 = _threefry2x32(key[0], key[1], hi, lo)
    return np.stack([b1, b2], axis=1)


def _np_uniform(key, n):
    bits = _random_bits32(key, n)
    fb = (bits >> np.uint32(9)) | np.uint32(0x3F800000)
    return fb.view(np.float32) - np.float32(1.0)


def _np_randint(key, n, span):
    ks = _np_split(key, 2)
    hi = _random_bits32(ks[0], n)
    lo = _random_bits32(ks[1], n)
    with np.errstate(over='ignore'):
        span_u = np.uint32(span)
        mult = (np.uint32(2 ** 16) % span_u)
        mult = (mult * mult) % span_u
        off = ((hi % span_u) * mult + lo % span_u) % span_u
    return off.astype(np.int32)


@functools.lru_cache(maxsize=1)
def _mask_constants():
    key = np.array([0, 1], np.uint32)
    k123 = _np_split(key, 3)
    rand1 = _np_uniform(k123[0], _E)
    rand2 = _np_uniform(k123[1], _E)
    rand_idx = _np_randint(k123[2], _E, _E)
    mask = rand1 < np.float32(_MASK_RATIO)
    use_token = mask & (rand2 < 0.8)
    use_rand = mask & (rand2 >= 0.8) & (rand2 < 0.9)
    row_idx = np.where(use_token, _E, np.where(use_rand, rand_idx,
                                               np.arange(_E)))
    return mask, row_idx.astype(np.int32)


def kernel(x, edge_index, edge_attr, params):
    mask_np, row_idx = _mask_constants()
    mask = jnp.asarray(mask_np)
    num_masked = jnp.sum(mask)
    mf = np.broadcast_to(mask_np.astype(np.float32)[:, None], (_E, _EIN))
    mf_j = jnp.asarray(np.ascontiguousarray(mf))

    src = edge_index[0]
    dst = edge_index[1]

    tbl = jnp.concatenate(
        [edge_attr,
         jnp.broadcast_to(params['mask_token'][None, :], (8, _EIN))],
        axis=0,
    )
    masked_ea = _sc_masked_gather(
        tbl, jnp.asarray(row_idx).reshape(_NW, _NCHUNK, _CH))

    zeros_n = jnp.zeros((_N, _H), jnp.float32)
    zeros_b = jnp.zeros((1, _H), jnp.float32)
    perm = jnp.asarray(_PERM)

    h = _tc_linear_relu(x, params['Wn'], params['bn'][None, :], 400)

    def pack32(t):
        return lax.bitcast_convert_type(
            t.reshape(_N, _H // 2, 2), jnp.int32)

    for l in range(_L):
        wm = params['Wm'][l]
        hs, hd = _tc_dual_mm(h, wm[:_H], wm[_H:2 * _H], zeros_b, 400)
        ee = _tc_edge_feat(masked_ea, params['We'], params['be'][None, :],
                           wm[2 * _H:][:, perm],
                           params['bm'][l][perm][None, :], 2000)
        aggp = _sc_message_pass(src, dst, pack32(hs), pack32(hd),
                                ee, zeros_n)
        wu = params['Wu'][l]
        h = _tc_update(h, aggp, wu[:_H], wu[_H:][perm, :],
                       params['bu'][l][None, :], 400)

    w1 = params['W1']
    ns, nd = _tc_dual_mm(h, w1[:_H], w1[_H:], params['b1'][None, :], 400,
                         dtype=jnp.float32)
    hid = _sc_decoder_gather(src, dst, ns, nd)
    recon, lacc = _tc_recon_loss(hid, edge_attr, mf_j, params['W2'],
                                 params['b2'][None, :], 2000)

    denom = jnp.maximum(num_masked.astype(jnp.float32) * _EIN, 1.0)
    loss = lacc[0, 0] / denom
    return recon, mask, edge_attr, loss, num_masked


# speedup vs baseline: 1.0542x; 1.0542x over previous
import functools

import numpy as np
import jax
import jax.numpy as jnp
from jax import lax
from jax.experimental import pallas as pl
from jax.experimental.pallas import tpu as pltpu
from jax.experimental.pallas import tpu_sc as plsc

_N = 10000
_E = 320000
_EIN = 16
_H = 128
_L = 4
_MASK_RATIO = 0.15

_NC = 2
_NS = 16
_NW = _NC * _NS
_EPW = _E // _NW
_CH = 80
_NCHUNK = _EPW // _CH
_CH1 = 40
_NCHUNK1 = _EPW // _CH1
_ETBL = _E + 4000
_RPT = (_N // _NS) // 8 * 8
_RTAIL = _N - _RPT * _NS

_PERM = (np.arange(16)[None, None, :] * 2
         + np.array([0, 1])[None, :, None]
         + (np.arange(_H // 32) * 32)[:, None, None]).reshape(_H)

_mesh = plsc.VectorSubcoreMesh(core_axis_name="c", subcore_axis_name="s")


@functools.partial(
    pl.kernel,
    out_type=jax.ShapeDtypeStruct((_E, _EIN), jnp.float32),
    mesh=_mesh,
    scratch_types=[
        pltpu.VMEM((_NCHUNK, _CH), jnp.int32),
        pltpu.VMEM((2, _CH, _EIN), jnp.float32),
        pltpu.SemaphoreType.DMA((2,)),
        pltpu.SemaphoreType.DMA((2,)),
    ],
    compiler_params=pltpu.CompilerParams(use_tc_tiling_on_sc=False),
)
def _sc_masked_gather(tbl, ridx3, out, idx_all, buf, gsem, wsem):
    c = lax.axis_index("c")
    s = lax.axis_index("s")
    wid = s * _NC + c
    base = wid * _EPW

    pltpu.sync_copy(ridx3.at[wid], idx_all)
    pltpu.async_copy(tbl.at[idx_all.at[0]], buf.at[0], gsem.at[0])

    def chunk(g, carry):
        par = lax.rem(g, 2)
        npar = lax.rem(g + 1, 2)

        @pl.when(g + 1 < _NCHUNK)
        def _():
            @pl.when(g >= 1)
            def _():
                pltpu.make_async_copy(
                    buf.at[npar], out.at[pl.ds(base, _CH)], wsem.at[npar]
                ).wait()

            pltpu.async_copy(tbl.at[idx_all.at[g + 1]], buf.at[npar],
                             gsem.at[npar])

        pltpu.make_async_copy(tbl.at[idx_all.at[g]], buf.at[par],
                              gsem.at[par]).wait()
        pltpu.async_copy(buf.at[par], out.at[pl.ds(base + g * _CH, _CH)],
                         wsem.at[par])
        return carry

    lax.fori_loop(0, _NCHUNK, chunk, 0)
    pltpu.make_async_copy(buf.at[0], out.at[pl.ds(base, _CH)],
                          wsem.at[0]).wait()
    pltpu.make_async_copy(buf.at[1], out.at[pl.ds(base, _CH)],
                          wsem.at[1]).wait()


def _widen(packed):
    lo = plsc.bitcast(packed << 16, jnp.float32)
    hi = plsc.bitcast(packed & jnp.int32(-65536), jnp.float32)
    return lo, hi


@functools.partial(
    pl.kernel,
    out_type=jax.ShapeDtypeStruct((_NC, _N, _H), jnp.float32),
    mesh=_mesh,
    scratch_types=[
        pltpu.VMEM((3, _CH), jnp.int32),
        pltpu.VMEM((3, _CH), jnp.int32),
        pltpu.VMEM((2, _CH, _H // 2), jnp.int32),
        pltpu.VMEM((2, _CH, _H // 2), jnp.int32),
        pltpu.VMEM((2, _CH, _H), jnp.float32),
        pltpu.VMEM_SHARED((_N, _H), jnp.float32),
        pltpu.SemaphoreType.DMA((2,)),
        pltpu.SemaphoreType.DMA((2,)),
        pltpu.SemaphoreType.DMA((3,)),
    ],
    compiler_params=pltpu.CompilerParams(use_tc_tiling_on_sc=False,
                                         needs_layout_passes=False),
)
def _sc_message_pass(src, dst, hs, hd, ee, zero, out,
                     sring, dring, bufa, bufb, bufc, acc,
                     gsem, ssem, rsem):
    c = lax.axis_index("c")
    s = lax.axis_index("s")
    wid = s * _NC + c
    base = wid * _EPW

    pltpu.sync_copy(src.at[pl.ds(base, _CH)], sring.at[0])
    pltpu.sync_copy(dst.at[pl.ds(base, _CH)], dring.at[0])

    pltpu.sync_copy(zero.at[pl.ds(s * _RPT, _RPT)],
                    acc.at[pl.ds(s * _RPT, _RPT)])

    @pl.when(s == _NS - 1)
    def _():
        pltpu.sync_copy(zero.at[pl.ds(_NS * _RPT, _RTAIL)],
                        acc.at[pl.ds(_NS * _RPT, _RTAIL)])

    plsc.subcore_barrier()

    def issue(g, par, slot):
        pltpu.async_copy(hs.at[sring.at[slot]], bufa.at[par], gsem.at[par])
        pltpu.async_copy(hd.at[dring.at[slot]], bufb.at[par], gsem.at[par])
        pltpu.async_copy(ee.at[pl.ds(base + g * _CH, _CH)], bufc.at[par],
                         gsem.at[par])

    def ring_load(g, slot):
        pltpu.async_copy(src.at[pl.ds(base + g * _CH, _CH)],
                         sring.at[slot], rsem.at[slot])
        pltpu.async_copy(dst.at[pl.ds(base + g * _CH, _CH)],
                         dring.at[slot], rsem.at[slot])

    issue(0, 0, 0)
    ring_load(1, 1)

    def chunk(g, carry):
        par = lax.rem(g, 2)
        npar = lax.rem(g + 1, 2)
        slot = lax.rem(g, 3)
        nslot = lax.rem(g + 1, 3)
        n2slot = lax.rem(g + 2, 3)

        @pl.when(g >= 1)
        def _():
            pltpu.make_async_copy(
                bufc.at[npar], acc.at[dring.at[nslot]], ssem.at[npar]
            ).wait()

        @pl.when(g + 1 < _NCHUNK)
        def _():
            pltpu.make_async_copy(src.at[pl.ds(base, _CH)],
                                  sring.at[nslot], rsem.at[nslot]).wait()
            pltpu.make_async_copy(dst.at[pl.ds(base, _CH)],
                                  dring.at[nslot], rsem.at[nslot]).wait()
            issue(g + 1, npar, nslot)

            @pl.when(g + 2 < _NCHUNK)
            def _():
                ring_load(g + 2, n2slot)

        pltpu.make_async_copy(hs.at[sring.at[slot]], bufa.at[par],
                              gsem.at[par]).wait()
        pltpu.make_async_copy(hd.at[dring.at[slot]], bufb.at[par],
                              gsem.at[par]).wait()
        pltpu.make_async_copy(ee.at[pl.ds(base, _CH)], bufc.at[par],
                              gsem.at[par]).wait()

        def row(i2, rcarry):
            for u in range(2):
                i = i2 * 2 + u
                for j in range(_H // 32):
                    a_lo, a_hi = _widen(bufa[par, i, pl.ds(16 * j, 16)])
                    b_lo, b_hi = _widen(bufb[par, i, pl.ds(16 * j, 16)])
                    slo = pl.ds(32 * j, 16)
                    shi = pl.ds(32 * j + 16, 16)
                    bufc[par, i, slo] = jnp.maximum(
                        a_lo + b_lo + bufc[par, i, slo], 0.0)
                    bufc[par, i, shi] = jnp.maximum(
                        a_hi + b_hi + bufc[par, i, shi], 0.0)
            return rcarry

        lax.fori_loop(0, _CH // 2, row, 0)
        pltpu.async_copy(bufc.at[par], acc.at[dring.at[slot]],
                         ssem.at[par], add=True)
        return carry

    lax.fori_loop(0, _NCHUNK, chunk, 0)

    pltpu.make_async_copy(bufc.at[(_NCHUNK - 1) % 2], acc.at[dring.at[0]],
                          ssem.at[(_NCHUNK - 1) % 2]).wait()
    plsc.subcore_barrier()

    pltpu.sync_copy(acc.at[pl.ds(s * _RPT, _RPT)],
                    out.at[c, pl.ds(s * _RPT, _RPT)])

    @pl.when(s == _NS - 1)
    def _():
        pltpu.sync_copy(acc.at[pl.ds(_NS * _RPT, _RTAIL)],
                        out.at[c, pl.ds(_NS * _RPT, _RTAIL)])


@functools.partial(
    pl.kernel,
    out_type=jax.ShapeDtypeStruct((_E, _H), jnp.float32),
    mesh=_mesh,
    scratch_types=[
        pltpu.VMEM((_EPW,), jnp.int32),
        pltpu.VMEM((_EPW,), jnp.int32),
        pltpu.VMEM((2, _CH, _H // 2), jnp.int32),
        pltpu.VMEM((2, _CH, _H // 2), jnp.int32),
        pltpu.VMEM((2, _CH, _H), jnp.float32),
        pltpu.SemaphoreType.DMA((2,)),
        pltpu.SemaphoreType.DMA((2,)),
    ],
    compiler_params=pltpu.CompilerParams(use_tc_tiling_on_sc=False,
                                         needs_layout_passes=False),
)
def _sc_decoder_gather(src, dst, ns, nd, out,
                       sidx_all, didx_all, bufa, bufb, bufo, gsem, wsem):
    c = lax.axis_index("c")
    s = lax.axis_index("s")
    wid = s * _NC + c
    base = wid * _EPW

    pltpu.sync_copy(src.at[pl.ds(base, _EPW)], sidx_all)
    pltpu.sync_copy(dst.at[pl.ds(base, _EPW)], didx_all)

    def issue(g, par):
        pltpu.async_copy(ns.at[sidx_all.at[pl.ds(g * _CH, _CH)]],
                         bufa.at[par], gsem.at[par])
        pltpu.async_copy(nd.at[didx_all.at[pl.ds(g * _CH, _CH)]],
                         bufb.at[par], gsem.at[par])

    issue(0, 0)

    def chunk(g, carry):
        par = lax.rem(g, 2)
        npar = lax.rem(g + 1, 2)

        @pl.when(g + 1 < _NCHUNK)
        def _():
            issue(g + 1, npar)

        pltpu.make_async_copy(ns.at[sidx_all.at[pl.ds(0, _CH)]],
                              bufa.at[par], gsem.at[par]).wait()
        pltpu.make_async_copy(nd.at[didx_all.at[pl.ds(0, _CH)]],
                              bufb.at[par], gsem.at[par]).wait()

        @pl.when(g >= 2)
        def _():
            pltpu.make_async_copy(bufo.at[par], out.at[pl.ds(base, _CH)],
                                  wsem.at[par]).wait()

        def row(i2, rcarry):
            for u in range(2):
                i = i2 * 2 + u
                for j in range(_H // 32):
                    a_lo, a_hi = _widen(bufa[par, i, pl.ds(16 * j, 16)])
                    b_lo, b_hi = _widen(bufb[par, i, pl.ds(16 * j, 16)])
                    bufo[par, i, pl.ds(32 * j, 16)] = jnp.maximum(
                        a_lo + b_lo, 0.0)
                    bufo[par, i, pl.ds(32 * j + 16, 16)] = jnp.maximum(
                        a_hi + b_hi, 0.0)
            return rcarry

        lax.fori_loop(0, _CH // 2, row, 0)
        pltpu.async_copy(bufo.at[par], out.at[pl.ds(base + g * _CH, _CH)],
                         wsem.at[par])
        return carry

    lax.fori_loop(0, _NCHUNK, chunk, 0)
    pltpu.make_async_copy(bufo.at[0], out.at[pl.ds(base, _CH)],
                          wsem.at[0]).wait()
    pltpu.make_async_copy(bufo.at[1], out.at[pl.ds(base, _CH)],
                          wsem.at[1]).wait()


def _linear_relu_body(x_ref, w_ref, b_ref, o_ref):
    acc = jnp.dot(x_ref[...], w_ref[...], preferred_element_type=jnp.float32)
    o_ref[...] = jnp.maximum(acc + b_ref[...], 0.0)


def _tc_linear_relu(x, w, b, bm):
    r, k = x.shape
    o = w.shape[1]
    return pl.pallas_call(
        _linear_relu_body,
        grid=(r // bm,),
        in_specs=[
            pl.BlockSpec((bm, k), lambda i: (i, 0)),
            pl.BlockSpec((k, o), lambda i: (0, 0)),
            pl.BlockSpec((1, o), lambda i: (0, 0)),
        ],
        out_specs=pl.BlockSpec((bm, o), lambda i: (i, 0)),
        out_shape=jax.ShapeDtypeStruct((r, o), jnp.float32),
    )(x, w, b)


def _dual_mm_body(x_ref, ws_ref, wd_ref, bs_ref, os_ref, od_ref):
    xb = x_ref[...]
    os_ref[...] = (
        jnp.dot(xb, ws_ref[...], preferred_element_type=jnp.float32)
        + bs_ref[...]
    ).astype(os_ref.dtype)
    od_ref[...] = jnp.dot(
        xb, wd_ref[...], preferred_element_type=jnp.float32
    ).astype(od_ref.dtype)


def _tc_dual_mm(x, ws, wd, bs, bm, dtype=jnp.bfloat16):
    r, k = x.shape
    o = ws.shape[1]
    return pl.pallas_call(
        _dual_mm_body,
        grid=(r // bm,),
        in_specs=[
            pl.BlockSpec((bm, k), lambda i: (i, 0)),
            pl.BlockSpec((k, o), lambda i: (0, 0)),
            pl.BlockSpec((k, o), lambda i: (0, 0)),
            pl.BlockSpec((1, o), lambda i: (0, 0)),
        ],
        out_specs=[
            pl.BlockSpec((bm, o), lambda i: (i, 0)),
            pl.BlockSpec((bm, o), lambda i: (i, 0)),
        ],
        out_shape=[
            jax.ShapeDtypeStruct((r, o), dtype),
            jax.ShapeDtypeStruct((r, o), dtype),
        ],
    )(x, ws, wd, bs)


def _encode_e_body(ma_ref, tok_ref, we_ref, be_ref, o_ref):
    nblk = _E // 2000
    ma = ma_ref[...]
    ma = jnp.where(pl.program_id(0) < nblk, ma,
                   jnp.broadcast_to(tok_ref[...], ma.shape))
    e = jnp.maximum(
        jnp.dot(ma, we_ref[...], preferred_element_type=jnp.float32)
        + be_ref[...],
        0.0,
    )
    o_ref[...] = e.astype(jnp.bfloat16)


def _tc_encode_e(ea, tok, we, be):
    bm = 2000
    nblk = _E // bm
    return pl.pallas_call(
        _encode_e_body,
        grid=(_ETBL // bm,),
        in_specs=[
            pl.BlockSpec((bm, _EIN), lambda i: (jnp.minimum(i, _E // 2000 - 1), 0)),
            pl.BlockSpec((1, _EIN), lambda i: (0, 0)),
            pl.BlockSpec((_EIN, _H), lambda i: (0, 0)),
            pl.BlockSpec((1, _H), lambda i: (0, 0)),
        ],
        out_specs=pl.BlockSpec((bm, _H), lambda i: (i, 0)),
        out_shape=jax.ShapeDtypeStruct((_ETBL, _H), jnp.bfloat16),
    )(ea, tok, we, be)


def _ee_mm_body(e_ref, wm_ref, bm_ref, o_ref):
    o_ref[...] = (
        jnp.dot(e_ref[...], wm_ref[...], preferred_element_type=jnp.float32)
        + bm_ref[...]
    )


def _tc_ee_mm(e, wm, bmb, bm):
    r = e.shape[0]
    return pl.pallas_call(
        _ee_mm_body,
        grid=(r // bm,),
        in_specs=[
            pl.BlockSpec((bm, _H), lambda i: (i, 0)),
            pl.BlockSpec((_H, _H), lambda i: (0, 0)),
            pl.BlockSpec((1, _H), lambda i: (0, 0)),
        ],
        out_specs=pl.BlockSpec((bm, _H), lambda i: (i, 0)),
        out_shape=jax.ShapeDtypeStruct((r, _H), jnp.float32),
    )(e, wm, bmb)


def _edge_feat_body(ma_ref, we_ref, be_ref, wm_ref, bm_ref, o_ref):
    e = jnp.maximum(
        jnp.dot(ma_ref[...], we_ref[...], preferred_element_type=jnp.float32)
        + be_ref[...],
        0.0,
    )
    o_ref[...] = (
        jnp.dot(e, wm_ref[...], preferred_element_type=jnp.float32)
        + bm_ref[...]
    )


def _tc_edge_feat(ma, we, be, wm, bmb, bm):
    r = ma.shape[0]
    return pl.pallas_call(
        _edge_feat_body,
        grid=(r // bm,),
        in_specs=[
            pl.BlockSpec((bm, _EIN), lambda i: (i, 0)),
            pl.BlockSpec((_EIN, _H), lambda i: (0, 0)),
            pl.BlockSpec((1, _H), lambda i: (0, 0)),
            pl.BlockSpec((_H, _H), lambda i: (0, 0)),
            pl.BlockSpec((1, _H), lambda i: (0, 0)),
        ],
        out_specs=pl.BlockSpec((bm, _H), lambda i: (i, 0)),
        out_shape=jax.ShapeDtypeStruct((r, _H), jnp.float32),
    )(ma, we, be, wm, bmb)


def _update_body(h_ref, a_ref, wh_ref, wa_ref, bu_ref, o_ref):
    hb = h_ref[...]
    ab = a_ref[0] + a_ref[1]
    upd = jnp.maximum(
        jnp.dot(hb, wh_ref[...], preferred_element_type=jnp.float32)
        + jnp.dot(ab, wa_ref[...], preferred_element_type=jnp.float32)
        + bu_ref[...],
        0.0,
    )
    o_ref[...] = hb + upd


def _tc_update(h, aggp, wh, wa, bu, bm):
    r = h.shape[0]
    return pl.pallas_call(
        _update_body,
        grid=(r // bm,),
        in_specs=[
            pl.BlockSpec((bm, _H), lambda i: (i, 0)),
            pl.BlockSpec((_NC, bm, _H), lambda i: (0, i, 0)),
            pl.BlockSpec((_H, _H), lambda i: (0, 0)),
            pl.BlockSpec((_H, _H), lambda i: (0, 0)),
            pl.BlockSpec((1, _H), lambda i: (0, 0)),
        ],
        out_specs=pl.BlockSpec((bm, _H), lambda i: (i, 0)),
        out_shape=jax.ShapeDtypeStruct((r, _H), jnp.float32),
    )(h, aggp, wh, wa, bu)


def _recon_body(hid_ref, ea_ref, mf_ref, w2_ref, b2_ref, rec_ref, lacc_ref):
    rec = (
        jnp.dot(hid_ref[...], w2_ref[...], preferred_element_type=jnp.float32)
        + b2_ref[...]
    )
    rec_ref[...] = rec
    d = (rec - ea_ref[...]) * mf_ref[...]
    part = jnp.sum(d * d)

    @pl.when(pl.program_id(0) == 0)
    def _():
        lacc_ref[0, 0] = 0.0

    lacc_ref[0, 0] += part


def _tc_recon_loss(hid, ea, mf, w2, b2, bm):
    r = hid.shape[0]
    return pl.pallas_call(
        _recon_body,
        grid=(r // bm,),
        in_specs=[
            pl.BlockSpec((bm, _H), lambda i: (i, 0)),
            pl.BlockSpec((bm, _EIN), lambda i: (i, 0)),
            pl.BlockSpec((bm, _EIN), lambda i: (i, 0)),
            pl.BlockSpec((_H, _EIN), lambda i: (0, 0)),
            pl.BlockSpec((1, _EIN), lambda i: (0, 0)),
        ],
        out_specs=[
            pl.BlockSpec((bm, _EIN), lambda i: (i, 0)),
            pl.BlockSpec(memory_space=pltpu.SMEM),
        ],
        out_shape=[
            jax.ShapeDtypeStruct((r, _EIN), jnp.float32),
            jax.ShapeDtypeStruct((1, 1), jnp.float32),
        ],
    )(hid, ea, mf, w2, b2)


_ROT0 = (13, 15, 26, 6)
_ROT1 = (17, 29, 16, 24)


def _rotl(x, d):
    return ((x << np.uint32(d)) | (x >> np.uint32(32 - d))).astype(np.uint32)


def _threefry2x32(k1, k2, x1, x2):
    with np.errstate(over='ignore'):
        ks = (np.uint32(k1), np.uint32(k2),
              np.uint32(k1) ^ np.uint32(k2) ^ np.uint32(0x1BD11BDA))
        x = [x1.astype(np.uint32) + ks[0], x2.astype(np.uint32) + ks[1]]
        rots = (_ROT0, _ROT1, _ROT0, _ROT1, _ROT0)
        kidx = ((1, 2), (2, 0), (0, 1), (1, 2), (2, 0))
        for i in range(5):
            for r in rots[i]:
                x[0] = (x[0] + x[1]).astype(np.uint32)
                x[1] = x[0] ^ _rotl(x[1], r)
            x[0] = (x[0] + ks[kidx[i][0]]).astype(np.uint32)
            x[1] = (x[1] + ks[kidx[i][1]] + np.uint32(i + 1)).astype(np.uint32)
    return x[0], x[1]


def _random_bits32(key, n):
    hi = np.zeros((n,), np.uint32)
    lo = np.arange(n, dtype=np.uint32)
    b1, b2 = _threefry2x32(key[0], key[1], hi, lo)
    return b1 ^ b2


def _np_split(key, num):
    hi = np.zeros((num,), np.uint32)
    lo = np.arange(num, dtype=np.uint32)
    b1, b2 = _threefry2x32(key[0], key[1], hi, lo)
    return np.stack([b1, b2], axis=1)


def _np_uniform(key, n):
    bits = _random_bits32(key, n)
    fb = (bits >> np.uint32(9)) | np.uint32(0x3F800000)
    return fb.view(np.float32) - np.float32(1.0)


def _np_randint(key, n, span):
    ks = _np_split(key, 2)
    hi = _random_bits32(ks[0], n)
    lo = _random_bits32(ks[1], n)
    with np.errstate(over='ignore'):
        span_u = np.uint32(span)
        mult = (np.uint32(2 ** 16) % span_u)
        mult = (mult * mult) % span_u
        off = ((hi % span_u) * mult + lo % span_u) % span_u
    return off.astype(np.int32)


@functools.lru_cache(maxsize=1)
def _mask_constants():
    key = np.array([0, 1], np.uint32)
    k123 = _np_split(key, 3)
    rand1 = _np_uniform(k123[0], _E)
    rand2 = _np_uniform(k123[1], _E)
    rand_idx = _np_randint(k123[2], _E, _E)
    mask = rand1 < np.float32(_MASK_RATIO)
    use_token = mask & (rand2 < 0.8)
    use_rand = mask & (rand2 >= 0.8) & (rand2 < 0.9)
    row_idx = np.where(use_token, _E, np.where(use_rand, rand_idx,
                                               np.arange(_E)))
    return mask, row_idx.astype(np.int32)


def kernel(x, edge_index, edge_attr, params):
    mask_np, row_idx = _mask_constants()
    mask = jnp.asarray(mask_np)
    num_masked = jnp.sum(mask)
    mf = np.broadcast_to(mask_np.astype(np.float32)[:, None], (_E, _EIN))
    mf_j = jnp.asarray(np.ascontiguousarray(mf))

    src = edge_index[0]
    dst = edge_index[1]

    tbl = jnp.concatenate(
        [edge_attr,
         jnp.broadcast_to(params['mask_token'][None, :], (8, _EIN))],
        axis=0,
    )
    masked_ea = _sc_masked_gather(
        tbl, jnp.asarray(row_idx).reshape(_NW, _NCHUNK, _CH))

    zeros_n = jnp.zeros((_N, _H), jnp.float32)
    zeros_b = jnp.zeros((1, _H), jnp.float32)
    perm = jnp.asarray(_PERM)

    h = _tc_linear_relu(x, params['Wn'], params['bn'][None, :], 400)

    def pack32(t):
        return lax.bitcast_convert_type(
            t.reshape(_N, _H // 2, 2), jnp.int32)

    for l in range(_L):
        wm = params['Wm'][l]
        hs, hd = _tc_dual_mm(h, wm[:_H], wm[_H:2 * _H], zeros_b, 400)
        ee = _tc_edge_feat(masked_ea, params['We'], params['be'][None, :],
                           wm[2 * _H:][:, perm],
                           params['bm'][l][perm][None, :], 2000)
        aggp = _sc_message_pass(src, dst, pack32(hs), pack32(hd),
                                ee, zeros_n)
        wu = params['Wu'][l]
        h = _tc_update(h, aggp, wu[:_H], wu[_H:][perm, :],
                       params['bu'][l][None, :], 400)

    w1 = params['W1']
    ns, nd = _tc_dual_mm(h, w1[:_H], w1[_H:], params['b1'][None, :], 400)
    hid = _sc_decoder_gather(src, dst, pack32(ns), pack32(nd))
    recon, lacc = _tc_recon_loss(hid, edge_attr, mf_j, params['W2'][perm, :],
                                 params['b2'][None, :], 2000)

    denom = jnp.maximum(num_masked.astype(jnp.float32) * _EIN, 1.0)
    loss = lacc[0, 0] / denom
    return recon, mask, edge_attr, loss, num_masked


# speedup vs baseline: 1.0548x; 1.0006x over previous
import functools

import numpy as np
import jax
import jax.numpy as jnp
from jax import lax
from jax.experimental import pallas as pl
from jax.experimental.pallas import tpu as pltpu
from jax.experimental.pallas import tpu_sc as plsc

_N = 10000
_E = 320000
_EIN = 16
_H = 128
_L = 4
_MASK_RATIO = 0.15

_NC = 2
_NS = 16
_NW = _NC * _NS
_EPW = _E // _NW
_CH = 80
_NCHUNK = _EPW // _CH
_RPT = (_N // _NS) // 8 * 8
_RTAIL = _N - _RPT * _NS

_PERM = (np.arange(16)[None, None, :] * 2
         + np.array([0, 1])[None, :, None]
         + (np.arange(_H // 32) * 32)[:, None, None]).reshape(_H)

_mesh = plsc.VectorSubcoreMesh(core_axis_name="c", subcore_axis_name="s")


@functools.partial(
    pl.kernel,
    out_type=jax.ShapeDtypeStruct((_E, _EIN), jnp.float32),
    mesh=_mesh,
    scratch_types=[
        pltpu.VMEM((_NCHUNK, _CH), jnp.int32),
        pltpu.VMEM((2, _CH, _EIN), jnp.float32),
        pltpu.SemaphoreType.DMA((2,)),
        pltpu.SemaphoreType.DMA((2,)),
    ],
    compiler_params=pltpu.CompilerParams(use_tc_tiling_on_sc=False),
)
def _sc_masked_gather(tbl, ridx3, out, idx_all, buf, gsem, wsem):
    c = lax.axis_index("c")
    s = lax.axis_index("s")
    wid = s * _NC + c
    base = wid * _EPW

    pltpu.sync_copy(ridx3.at[wid], idx_all)
    pltpu.async_copy(tbl.at[idx_all.at[0]], buf.at[0], gsem.at[0])

    def chunk(g, carry):
        par = lax.rem(g, 2)
        npar = lax.rem(g + 1, 2)

        @pl.when(g + 1 < _NCHUNK)
        def _():
            @pl.when(g >= 1)
            def _():
                pltpu.make_async_copy(
                    buf.at[npar], out.at[pl.ds(base, _CH)], wsem.at[npar]
                ).wait()

            pltpu.async_copy(tbl.at[idx_all.at[g + 1]], buf.at[npar],
                             gsem.at[npar])

        pltpu.make_async_copy(tbl.at[idx_all.at[g]], buf.at[par],
                              gsem.at[par]).wait()
        pltpu.async_copy(buf.at[par], out.at[pl.ds(base + g * _CH, _CH)],
                         wsem.at[par])
        return carry

    lax.fori_loop(0, _NCHUNK, chunk, 0)
    pltpu.make_async_copy(buf.at[0], out.at[pl.ds(base, _CH)],
                          wsem.at[0]).wait()
    pltpu.make_async_copy(buf.at[1], out.at[pl.ds(base, _CH)],
                          wsem.at[1]).wait()


def _widen(packed):
    lo = plsc.bitcast(packed << 16, jnp.float32)
    hi = plsc.bitcast(packed & jnp.int32(-65536), jnp.float32)
    return lo, hi


@functools.partial(
    pl.kernel,
    out_type=jax.ShapeDtypeStruct((_NC, _N, _H), jnp.float32),
    mesh=_mesh,
    scratch_types=[
        pltpu.VMEM((3, _CH), jnp.int32),
        pltpu.VMEM((3, _CH), jnp.int32),
        pltpu.VMEM((2, _CH, _H // 2), jnp.int32),
        pltpu.VMEM((2, _CH, _H // 2), jnp.int32),
        pltpu.VMEM((2, _CH, _H), jnp.float32),
        pltpu.VMEM_SHARED((_N, _H), jnp.float32),
        pltpu.SemaphoreType.DMA((2,)),
        pltpu.SemaphoreType.DMA((2,)),
        pltpu.SemaphoreType.DMA((3,)),
    ],
    compiler_params=pltpu.CompilerParams(use_tc_tiling_on_sc=False,
                                         needs_layout_passes=False),
)
def _sc_message_pass(src, dst, hs, hd, ee, zero, out,
                     sring, dring, bufa, bufb, bufc, acc,
                     gsem, ssem, rsem):
    c = lax.axis_index("c")
    s = lax.axis_index("s")
    wid = s * _NC + c
    base = wid * _EPW

    pltpu.sync_copy(src.at[pl.ds(base, _CH)], sring.at[0])
    pltpu.sync_copy(dst.at[pl.ds(base, _CH)], dring.at[0])

    pltpu.sync_copy(zero.at[pl.ds(s * _RPT, _RPT)],
                    acc.at[pl.ds(s * _RPT, _RPT)])

    @pl.when(s == _NS - 1)
    def _():
        pltpu.sync_copy(zero.at[pl.ds(_NS * _RPT, _RTAIL)],
                        acc.at[pl.ds(_NS * _RPT, _RTAIL)])

    plsc.subcore_barrier()

    def issue(g, par, slot):
        pltpu.async_copy(hs.at[sring.at[slot]], bufa.at[par], gsem.at[par])
        pltpu.async_copy(hd.at[dring.at[slot]], bufb.at[par], gsem.at[par])
        pltpu.async_copy(ee.at[pl.ds(base + g * _CH, _CH)], bufc.at[par],
                         gsem.at[par])

    def ring_load(g, slot):
        pltpu.async_copy(src.at[pl.ds(base + g * _CH, _CH)],
                         sring.at[slot], rsem.at[slot])
        pltpu.async_copy(dst.at[pl.ds(base + g * _CH, _CH)],
                         dring.at[slot], rsem.at[slot])

    issue(0, 0, 0)
    ring_load(1, 1)

    def chunk(g, carry):
        par = lax.rem(g, 2)
        npar = lax.rem(g + 1, 2)
        slot = lax.rem(g, 3)
        nslot = lax.rem(g + 1, 3)
        n2slot = lax.rem(g + 2, 3)

        @pl.when(g >= 1)
        def _():
            pltpu.make_async_copy(
                bufc.at[npar], acc.at[dring.at[nslot]], ssem.at[npar]
            ).wait()

        @pl.when(g + 1 < _NCHUNK)
        def _():
            pltpu.make_async_copy(src.at[pl.ds(base, _CH)],
                                  sring.at[nslot], rsem.at[nslot]).wait()
            pltpu.make_async_copy(dst.at[pl.ds(base, _CH)],
                                  dring.at[nslot], rsem.at[nslot]).wait()
            issue(g + 1, npar, nslot)

            @pl.when(g + 2 < _NCHUNK)
            def _():
                ring_load(g + 2, n2slot)

        pltpu.make_async_copy(hs.at[sring.at[slot]], bufa.at[par],
                              gsem.at[par]).wait()
        pltpu.make_async_copy(hd.at[dring.at[slot]], bufb.at[par],
                              gsem.at[par]).wait()
        pltpu.make_async_copy(ee.at[pl.ds(base, _CH)], bufc.at[par],
                              gsem.at[par]).wait()

        def row(i2, rcarry):
            for u in range(2):
                i = i2 * 2 + u
                for j in range(_H // 32):
                    a_lo, a_hi = _widen(bufa[par, i, pl.ds(16 * j, 16)])
                    b_lo, b_hi = _widen(bufb[par, i, pl.ds(16 * j, 16)])
                    slo = pl.ds(32 * j, 16)
                    shi = pl.ds(32 * j + 16, 16)
                    bufc[par, i, slo] = jnp.maximum(
                        a_lo + b_lo + bufc[par, i, slo], 0.0)
                    bufc[par, i, shi] = jnp.maximum(
                        a_hi + b_hi + bufc[par, i, shi], 0.0)
            return rcarry

        lax.fori_loop(0, _CH // 2, row, 0)
        pltpu.async_copy(bufc.at[par], acc.at[dring.at[slot]],
                         ssem.at[par], add=True)
        return carry

    lax.fori_loop(0, _NCHUNK, chunk, 0)

    pltpu.make_async_copy(bufc.at[(_NCHUNK - 1) % 2], acc.at[dring.at[0]],
                          ssem.at[(_NCHUNK - 1) % 2]).wait()
    plsc.subcore_barrier()

    pltpu.sync_copy(acc.at[pl.ds(s * _RPT, _RPT)],
                    out.at[c, pl.ds(s * _RPT, _RPT)])

    @pl.when(s == _NS - 1)
    def _():
        pltpu.sync_copy(acc.at[pl.ds(_NS * _RPT, _RTAIL)],
                        out.at[c, pl.ds(_NS * _RPT, _RTAIL)])


@functools.partial(
    pl.kernel,
    out_type=jax.ShapeDtypeStruct((_E, _H), jnp.float32),
    mesh=_mesh,
    scratch_types=[
        pltpu.VMEM((_EPW,), jnp.int32),
        pltpu.VMEM((_EPW,), jnp.int32),
        pltpu.VMEM((2, _CH, _H // 2), jnp.int32),
        pltpu.VMEM((2, _CH, _H // 2), jnp.int32),
        pltpu.VMEM((2, _CH, _H), jnp.float32),
        pltpu.SemaphoreType.DMA((2,)),
        pltpu.SemaphoreType.DMA((2,)),
    ],
    compiler_params=pltpu.CompilerParams(use_tc_tiling_on_sc=False,
                                         needs_layout_passes=False),
)
def _sc_decoder_gather(src, dst, ns, nd, out,
                       sidx_all, didx_all, bufa, bufb, bufo, gsem, wsem):
    c = lax.axis_index("c")
    s = lax.axis_index("s")
    wid = s * _NC + c
    base = wid * _EPW

    pltpu.sync_copy(src.at[pl.ds(base, _EPW)], sidx_all)
    pltpu.sync_copy(dst.at[pl.ds(base, _EPW)], didx_all)

    def issue(g, par):
        pltpu.async_copy(ns.at[sidx_all.at[pl.ds(g * _CH, _CH)]],
                         bufa.at[par], gsem.at[par])
        pltpu.async_copy(nd.at[didx_all.at[pl.ds(g * _CH, _CH)]],
                         bufb.at[par], gsem.at[par])

    issue(0, 0)

    def chunk(g, carry):
        par = lax.rem(g, 2)
        npar = lax.rem(g + 1, 2)

        @pl.when(g + 1 < _NCHUNK)
        def _():
            issue(g + 1, npar)

        pltpu.make_async_copy(ns.at[sidx_all.at[pl.ds(0, _CH)]],
                              bufa.at[par], gsem.at[par]).wait()
        pltpu.make_async_copy(nd.at[didx_all.at[pl.ds(0, _CH)]],
                              bufb.at[par], gsem.at[par]).wait()

        @pl.when(g >= 2)
        def _():
            pltpu.make_async_copy(bufo.at[par], out.at[pl.ds(base, _CH)],
                                  wsem.at[par]).wait()

        def row(i2, rcarry):
            for u in range(2):
                i = i2 * 2 + u
                for j in range(_H // 32):
                    a_lo, a_hi = _widen(bufa[par, i, pl.ds(16 * j, 16)])
                    b_lo, b_hi = _widen(bufb[par, i, pl.ds(16 * j, 16)])
                    bufo[par, i, pl.ds(32 * j, 16)] = jnp.maximum(
                        a_lo + b_lo, 0.0)
                    bufo[par, i, pl.ds(32 * j + 16, 16)] = jnp.maximum(
                        a_hi + b_hi, 0.0)
            return rcarry

        lax.fori_loop(0, _CH // 2, row, 0)
        pltpu.async_copy(bufo.at[par], out.at[pl.ds(base + g * _CH, _CH)],
                         wsem.at[par])
        return carry

    lax.fori_loop(0, _NCHUNK, chunk, 0)
    pltpu.make_async_copy(bufo.at[0], out.at[pl.ds(base, _CH)],
                          wsem.at[0]).wait()
    pltpu.make_async_copy(bufo.at[1], out.at[pl.ds(base, _CH)],
                          wsem.at[1]).wait()


def _linear_relu_body(x_ref, w_ref, b_ref, o_ref):
    acc = jnp.dot(x_ref[...], w_ref[...], preferred_element_type=jnp.float32)
    o_ref[...] = jnp.maximum(acc + b_ref[...], 0.0)


def _tc_linear_relu(x, w, b, bm):
    r, k = x.shape
    o = w.shape[1]
    return pl.pallas_call(
        _linear_relu_body,
        grid=(r // bm,),
        in_specs=[
            pl.BlockSpec((bm, k), lambda i: (i, 0)),
            pl.BlockSpec((k, o), lambda i: (0, 0)),
            pl.BlockSpec((1, o), lambda i: (0, 0)),
        ],
        out_specs=pl.BlockSpec((bm, o), lambda i: (i, 0)),
        out_shape=jax.ShapeDtypeStruct((r, o), jnp.float32),
    )(x, w, b)


def _dual_mm_body(x_ref, ws_ref, wd_ref, bs_ref, os_ref, od_ref):
    xb = x_ref[...]
    os_ref[...] = (
        jnp.dot(xb, ws_ref[...], preferred_element_type=jnp.float32)
        + bs_ref[...]
    ).astype(os_ref.dtype)
    od_ref[...] = jnp.dot(
        xb, wd_ref[...], preferred_element_type=jnp.float32
    ).astype(od_ref.dtype)


def _tc_dual_mm(x, ws, wd, bs, bm, dtype=jnp.bfloat16):
    r, k = x.shape
    o = ws.shape[1]
    return pl.pallas_call(
        _dual_mm_body,
        grid=(r // bm,),
        in_specs=[
            pl.BlockSpec((bm, k), lambda i: (i, 0)),
            pl.BlockSpec((k, o), lambda i: (0, 0)),
            pl.BlockSpec((k, o), lambda i: (0, 0)),
            pl.BlockSpec((1, o), lambda i: (0, 0)),
        ],
        out_specs=[
            pl.BlockSpec((bm, o), lambda i: (i, 0)),
            pl.BlockSpec((bm, o), lambda i: (i, 0)),
        ],
        out_shape=[
            jax.ShapeDtypeStruct((r, o), dtype),
            jax.ShapeDtypeStruct((r, o), dtype),
        ],
    )(x, ws, wd, bs)


def _edge_feat_body(ma_ref, we_ref, be_ref, wm_ref, bm_ref, o_ref):
    e = jnp.maximum(
        jnp.dot(ma_ref[...], we_ref[...], preferred_element_type=jnp.float32)
        + be_ref[...],
        0.0,
    )
    o_ref[...] = (
        jnp.dot(e, wm_ref[...], preferred_element_type=jnp.float32)
        + bm_ref[...]
    )


def _tc_edge_feat(ma, we, be, wm, bmb, bm):
    r = ma.shape[0]
    return pl.pallas_call(
        _edge_feat_body,
        grid=(r // bm,),
        in_specs=[
            pl.BlockSpec((bm, _EIN), lambda i: (i, 0)),
            pl.BlockSpec((_EIN, _H), lambda i: (0, 0)),
            pl.BlockSpec((1, _H), lambda i: (0, 0)),
            pl.BlockSpec((_H, _H), lambda i: (0, 0)),
            pl.BlockSpec((1, _H), lambda i: (0, 0)),
        ],
        out_specs=pl.BlockSpec((bm, _H), lambda i: (i, 0)),
        out_shape=jax.ShapeDtypeStruct((r, _H), jnp.float32),
    )(ma, we, be, wm, bmb)


def _update_body(h_ref, a_ref, wh_ref, wa_ref, bu_ref, o_ref):
    hb = h_ref[...]
    ab = a_ref[0] + a_ref[1]
    upd = jnp.maximum(
        jnp.dot(hb, wh_ref[...], preferred_element_type=jnp.float32)
        + jnp.dot(ab, wa_ref[...], preferred_element_type=jnp.float32)
        + bu_ref[...],
        0.0,
    )
    o_ref[...] = hb + upd


def _tc_update(h, aggp, wh, wa, bu, bm):
    r = h.shape[0]
    return pl.pallas_call(
        _update_body,
        grid=(r // bm,),
        in_specs=[
            pl.BlockSpec((bm, _H), lambda i: (i, 0)),
            pl.BlockSpec((_NC, bm, _H), lambda i: (0, i, 0)),
            pl.BlockSpec((_H, _H), lambda i: (0, 0)),
            pl.BlockSpec((_H, _H), lambda i: (0, 0)),
            pl.BlockSpec((1, _H), lambda i: (0, 0)),
        ],
        out_specs=pl.BlockSpec((bm, _H), lambda i: (i, 0)),
        out_shape=jax.ShapeDtypeStruct((r, _H), jnp.float32),
    )(h, aggp, wh, wa, bu)


def _recon_body(hid_ref, ea_ref, mf_ref, w2_ref, b2_ref, rec_ref, lacc_ref):
    rec = (
        jnp.dot(hid_ref[...], w2_ref[...], preferred_element_type=jnp.float32)
        + b2_ref[...]
    )
    rec_ref[...] = rec
    d = (rec - ea_ref[...]) * mf_ref[...]
    part = jnp.sum(d * d)

    @pl.when(pl.program_id(0) == 0)
    def _():
        lacc_ref[0, 0] = 0.0

    lacc_ref[0, 0] += part


def _tc_recon_loss(hid, ea, mf, w2, b2, bm):
    r = hid.shape[0]
    return pl.pallas_call(
        _recon_body,
        grid=(r // bm,),
        in_specs=[
            pl.BlockSpec((bm, _H), lambda i: (i, 0)),
            pl.BlockSpec((bm, _EIN), lambda i: (i, 0)),
            pl.BlockSpec((bm, _EIN), lambda i: (i, 0)),
            pl.BlockSpec((_H, _EIN), lambda i: (0, 0)),
            pl.BlockSpec((1, _EIN), lambda i: (0, 0)),
        ],
        out_specs=[
            pl.BlockSpec((bm, _EIN), lambda i: (i, 0)),
            pl.BlockSpec(memory_space=pltpu.SMEM),
        ],
        out_shape=[
            jax.ShapeDtypeStruct((r, _EIN), jnp.float32),
            jax.ShapeDtypeStruct((1, 1), jnp.float32),
        ],
    )(hid, ea, mf, w2, b2)


_ROT0 = (13, 15, 26, 6)
_ROT1 = (17, 29, 16, 24)


def _rotl(x, d):
    return ((x << np.uint32(d)) | (x >> np.uint32(32 - d))).astype(np.uint32)


def _threefry2x32(k1, k2, x1, x2):
    with np.errstate(over='ignore'):
        ks = (np.uint32(k1), np.uint32(k2),
              np.uint32(k1) ^ np.uint32(k2) ^ np.uint32(0x1BD11BDA))
        x = [x1.astype(np.uint32) + ks[0], x2.astype(np.uint32) + ks[1]]
        rots = (_ROT0, _ROT1, _ROT0, _ROT1, _ROT0)
        kidx = ((1, 2), (2, 0), (0, 1), (1, 2), (2, 0))
        for i in range(5):
            for r in rots[i]:
                x[0] = (x[0] + x[1]).astype(np.uint32)
                x[1] = x[0] ^ _rotl(x[1], r)
            x[0] = (x[0] + ks[kidx[i][0]]).astype(np.uint32)
            x[1] = (x[1] + ks[kidx[i][1]] + np.uint32(i + 1)).astype(np.uint32)
    return x[0], x[1]


def _random_bits32(key, n):
    hi = np.zeros((n,), np.uint32)
    lo = np.arange(n, dtype=np.uint32)
    b1, b2 = _threefry2x32(key[0], key[1], hi, lo)
    return b1 ^ b2


def _np_split(key, num):
    hi = np.zeros((num,), np.uint32)
    lo = np.arange(num, dtype=np.uint32)
    b1, b2 = _threefry2x32(key[0], key[1], hi, lo)
    return np.stack([b1, b2], axis=1)


def _np_uniform(key, n):
    bits = _random_bits32(key, n)
    fb = (bits >> np.uint32(9)) | np.uint32(0x3F800000)
    return fb.view(np.float32) - np.float32(1.0)


def _np_randint(key, n, span):
    ks = _np_split(key, 2)
    hi = _random_bits32(ks[0], n)
    lo = _random_bits32(ks[1], n)
    with np.errstate(over='ignore'):
        span_u = np.uint32(span)
        mult = (np.uint32(2 ** 16) % span_u)
        mult = (mult * mult) % span_u
        off = ((hi % span_u) * mult + lo % span_u) % span_u
    return off.astype(np.int32)


@functools.lru_cache(maxsize=1)
def _mask_constants():
    key = np.array([0, 1], np.uint32)
    k123 = _np_split(key, 3)
    rand1 = _np_uniform(k123[0], _E)
    rand2 = _np_uniform(k123[1], _E)
    rand_idx = _np_randint(k123[2], _E, _E)
    mask = rand1 < np.float32(_MASK_RATIO)
    use_token = mask & (rand2 < 0.8)
    use_rand = mask & (rand2 >= 0.8) & (rand2 < 0.9)
    row_idx = np.where(use_token, _E, np.where(use_rand, rand_idx,
                                               np.arange(_E)))
    return mask, row_idx.astype(np.int32)


def kernel(x, edge_index, edge_attr, params):
    mask_np, row_idx = _mask_constants()
    mask = jnp.asarray(mask_np)
    num_masked = jnp.sum(mask)
    mf = np.broadcast_to(mask_np.astype(np.float32)[:, None], (_E, _EIN))
    mf_j = jnp.asarray(np.ascontiguousarray(mf))

    src = edge_index[0]
    dst = edge_index[1]

    tbl = jnp.concatenate(
        [edge_attr,
         jnp.broadcast_to(params['mask_token'][None, :], (8, _EIN))],
        axis=0,
    )
    masked_ea = _sc_masked_gather(
        tbl, jnp.asarray(row_idx).reshape(_NW, _NCHUNK, _CH))

    zeros_n = jnp.zeros((_N, _H), jnp.float32)
    zeros_b = jnp.zeros((1, _H), jnp.float32)
    perm = jnp.asarray(_PERM)

    h = _tc_linear_relu(x, params['Wn'], params['bn'][None, :], 400)

    def pack32(t):
        return lax.bitcast_convert_type(
            t.reshape(_N, _H // 2, 2), jnp.int32)

    for l in range(_L):
        wm = params['Wm'][l]
        hs, hd = _tc_dual_mm(h, wm[:_H], wm[_H:2 * _H], zeros_b, 400)
        ee = _tc_edge_feat(masked_ea, params['We'], params['be'][None, :],
                           wm[2 * _H:][:, perm],
                           params['bm'][l][perm][None, :], 2000)
        aggp = _sc_message_pass(src, dst, pack32(hs), pack32(hd),
                                ee, zeros_n)
        wu = params['Wu'][l]
        h = _tc_update(h, aggp, wu[:_H], wu[_H:][perm, :],
                       params['bu'][l][None, :], 400)

    w1 = params['W1']
    ns, nd = _tc_dual_mm(h, w1[:_H], w1[_H:], params['b1'][None, :], 400)
    hid = _sc_decoder_gather(src, dst, pack32(ns), pack32(nd))
    recon, lacc = _tc_recon_loss(hid, edge_attr, mf_j, params['W2'][perm, :],
                                 params['b2'][None, :], 2000)

    denom = jnp.maximum(num_masked.astype(jnp.float32) * _EIN, 1.0)
    loss = lacc[0, 0] / denom
    return recon, mask, edge_attr, loss, num_masked
